# baseline (speedup 1.0000x reference)
import functools

import jax
import jax.numpy as jnp
from jax import lax
from jax.experimental import pallas as pl
from jax.experimental.pallas import tpu as pltpu
from jax.experimental.pallas import tpu_sc as plsc

LANES = 16
NBIG = 2048
NSMALL = 256
STRIDE = 3 * NBIG
HISTW = LANES * STRIDE
NELEM = 512 * 512
HALFN = NELEM // 2
CHUNK = 4096
NCHUNK = HALFN // CHUNK
R25 = NELEM // 4 + 1
R50 = NELEM // 2 + 1
R75 = (3 * NELEM) // 4 + 1
UNROLL = 8


def _luma_body(p_ref, t_ref, lp_ref, lt_ref, sp_ref, st_ref):
    p0 = p_ref[0, 0]
    p1 = p_ref[0, 1]
    p2 = p_ref[0, 2]
    lp_ref[0] = 0.299 * p0 + 0.587 * p1 + 0.114 * p2
    sp = jnp.sum(p0) + jnp.sum(p1) + jnp.sum(p2)
    sp_ref[0, 0, :] = jnp.full((128,), sp, jnp.float32)
    t0 = t_ref[0, 0]
    t1 = t_ref[0, 1]
    t2 = t_ref[0, 2]
    lt_ref[0] = 0.299 * t0 + 0.587 * t1 + 0.114 * t2
    st = jnp.sum(t0) + jnp.sum(t1) + jnp.sum(t2)
    st_ref[0, 0, :] = jnp.full((128,), st, jnp.float32)


def _sc_select_body(lp_hbm, lt_hbm, out_hbm, buf0, buf1, hist, myflat, pflat,
                    vtmp, sh_hist, sem0, sem1):
    c = lax.axis_index("c")
    s = lax.axis_index("s")
    img = c * 8 + (s >> 1)
    half = s & 1
    part = s ^ 1
    lanes = lax.iota(jnp.int32, 16)
    ones = jnp.ones((16,), jnp.int32)
    zeros16 = jnp.zeros((16,), jnp.int32)
    lanebase = lanes * STRIDE
    base = half * HALFN

    def start_dma(ci, buf, sem):
        off = base + ci * CHUNK

        @pl.when(img < 8)
        def _():
            pltpu.async_copy(lp_hbm.at[img, pl.ds(off, CHUNK)], buf, sem)

        @pl.when(img >= 8)
        def _():
            pltpu.async_copy(lt_hbm.at[img - 8, pl.ds(off, CHUNK)], buf, sem)

    def wait_dma(buf, sem):
        pltpu.make_async_copy(lp_hbm.at[0, pl.ds(0, CHUNK)], buf, sem).wait()

    def stream_pass(per_vreg, car):
        car = jax.tree.map(jnp.asarray, car)
        def chunk_process(buf, c):
            def body(i, c2):
                x = buf[pl.ds(i, 16)]
                return per_vreg(x, c2)
            return plsc.parallel_loop(0, CHUNK, 16, unroll=UNROLL,
                                      carry=c)(body)

        def pair_body(i, c):
            ci0 = 2 * i
            wait_dma(buf0, sem0)
            start_dma(ci0 + 1, buf1, sem1)
            c = chunk_process(buf0, c)
            wait_dma(buf1, sem1)

            @pl.when(ci0 + 2 < NCHUNK)
            def _():
                start_dma(ci0 + 2, buf0, sem0)

            c = chunk_process(buf1, c)
            return c

        start_dma(0, buf0, sem0)
        return lax.fori_loop(0, NCHUNK // 2, pair_body, car)

    def zero_hist():
        def body(i):
            hist[pl.ds(i, 16)] = zeros16
        plsc.parallel_loop(0, HISTW, 16, unroll=8)(body)

    def zero_cols(ncols_pow2):
        sh = ncols_pow2.bit_length() - 1
        msk = ncols_pow2 - 1

        def body(i):
            hist[pl.ds(((i >> sh) * STRIDE) + (i & msk), 16)] = zeros16
        plsc.parallel_loop(0, LANES * ncols_pow2, 16, unroll=8)(body)

    def lane_reduce(ncols):
        def body(cv):
            acc = zeros16
            for l in range(LANES):
                acc = acc + hist[pl.ds(l * STRIDE + cv, 16)]
            myflat[pl.ds(cv, 16)] = acc
        plsc.parallel_loop(0, ncols, 16, unroll=2)(body)

    def publish(ncols):
        pltpu.sync_copy(myflat.at[pl.ds(0, ncols)],
                        sh_hist.at[s, pl.ds(0, ncols)])

    def fetch(ncols):
        pltpu.sync_copy(sh_hist.at[part, pl.ds(0, ncols)],
                        pflat.at[pl.ds(0, ncols)])

    def find(nbins, segoff, rank):
        def body(i, car):
            cum, b, cb = car
            o = segoff + i * 16
            v = myflat[pl.ds(o, 16)] + pflat[pl.ds(o, 16)]
            pc = plsc.cumsum(v) + cum
            lt = pc < rank
            b = b + jnp.sum(jnp.where(lt, ones, zeros16))
            cb = jnp.maximum(cb, jnp.max(jnp.where(lt, pc, zeros16)))
            return jnp.max(pc), b, cb
        z = jnp.int32(0)
        _, b, cb = lax.fori_loop(0, nbins // 16, body, (z, z, z))
        return b, cb

    zero_cols(NBIG)

    def p1_vreg(x, car2):
        vmin, vmax = car2
        bits = lax.bitcast_convert_type(x, jnp.int32)
        plsc.addupdate_scatter(hist, [lanebase + (bits >> 19)], ones)
        return jnp.minimum(vmin, x), jnp.maximum(vmax, x)

    vmin0 = jnp.full((16,), jnp.inf, jnp.float32)
    vmax0 = jnp.full((16,), -jnp.inf, jnp.float32)
    vmin, vmax = stream_pass(p1_vreg, (vmin0, vmax0))
    mn = jnp.min(vmin)
    mx = jnp.max(vmax)

    lane_reduce(NBIG)
    mmv = jnp.where(lanes == 1, mx, mn)
    myflat[pl.ds(NBIG, 16)] = lax.bitcast_convert_type(mmv, jnp.int32)
    publish(NBIG + 128)
    plsc.subcore_barrier()
    fetch(NBIG + 128)
    pmm = lax.bitcast_convert_type(pflat[pl.ds(NBIG, 16)], jnp.float32)
    mn = jnp.minimum(mn, jnp.min(pmm))
    mx = jnp.maximum(mx, jnp.max(pmm))
    plsc.subcore_barrier()

    b25, c25 = find(NBIG, 0, jnp.int32(R25))
    b50, c50 = find(NBIG, 0, jnp.int32(R50))
    b75, c75 = find(NBIG, 0, jnp.int32(R75))
    r2_25 = jnp.int32(R25) - c25
    r2_50 = jnp.int32(R50) - c50
    r2_75 = jnp.int32(R75) - c75

    zero_hist()

    def p2_vreg(x, car2):
        bits = lax.bitcast_convert_type(x, jnp.int32)
        top = bits >> 19
        m25 = top == b25
        m50 = top == b50
        m75 = top == b75
        sel = jnp.where(m25, 0, jnp.where(m50, NBIG, 2 * NBIG))
        addr = lanebase + sel + ((bits >> 8) & (NBIG - 1))
        plsc.addupdate_scatter(hist, [addr], ones, mask=m25 | m50 | m75)
        return car2

    stream_pass(p2_vreg, 0)
    lane_reduce(3 * NBIG)
    publish(3 * NBIG)
    plsc.subcore_barrier()
    fetch(3 * NBIG)
    plsc.subcore_barrier()
    b2_25, c2_25 = find(NBIG, 0, r2_25)
    b2_50, c2_50 = find(NBIG, NBIG, r2_50)
    b2_75, c2_75 = find(NBIG, 2 * NBIG, r2_75)
    r3_25 = r2_25 - c2_25
    r3_50 = r2_50 - c2_50
    r3_75 = r2_75 - c2_75
    pre25 = (b25 << 11) | b2_25
    pre50 = (b50 << 11) | b2_50
    pre75 = (b75 << 11) | b2_75

    zero_cols(4 * NSMALL)

    def p3_vreg(x, car2):
        bits = lax.bitcast_convert_type(x, jnp.int32)
        hi = bits >> 8
        m25 = hi == pre25
        m50 = hi == pre50
        m75 = hi == pre75
        sel = jnp.where(m25, 0, jnp.where(m50, NSMALL, 2 * NSMALL))
        addr = lanebase + sel + (bits & (NSMALL - 1))
        plsc.addupdate_scatter(hist, [addr], ones, mask=m25 | m50 | m75)
        return car2

    stream_pass(p3_vreg, 0)
    lane_reduce(3 * NSMALL)
    publish(3 * NSMALL)
    plsc.subcore_barrier()
    fetch(3 * NSMALL)
    b3_25, _ = find(NSMALL, 0, r3_25)
    b3_50, _ = find(NSMALL, NSMALL, r3_50)
    b3_75, _ = find(NSMALL, 2 * NSMALL, r3_75)

    v25 = lax.bitcast_convert_type(
        jnp.broadcast_to((b25 << 19) | (b2_25 << 8) | b3_25, (16,)),
        jnp.float32)
    v50 = lax.bitcast_convert_type(
        jnp.broadcast_to((b50 << 19) | (b2_50 << 8) | b3_50, (16,)),
        jnp.float32)
    v75 = lax.bitcast_convert_type(
        jnp.broadcast_to((b75 << 19) | (b2_75 << 8) | b3_75, (16,)),
        jnp.float32)

    outv = jnp.where(lanes == 0, mn, jnp.zeros((16,), jnp.float32))
    outv = jnp.where(lanes == 1, v25, outv)
    outv = jnp.where(lanes == 2, v50, outv)
    outv = jnp.where(lanes == 3, v75, outv)
    outv = jnp.where(lanes == 4, mx, outv)
    vtmp[...] = outv

    @pl.when(half == 0)
    def _():
        pltpu.sync_copy(vtmp, out_hbm.at[img])


def _select_call(luma_p2, luma_t2):
    sel = functools.partial(
        pl.kernel,
        out_type=jax.ShapeDtypeStruct((16, 16), jnp.float32),
        mesh=plsc.VectorSubcoreMesh(core_axis_name="c", subcore_axis_name="s"),
        compiler_params=pltpu.CompilerParams(needs_layout_passes=False),
        scratch_types=[
            pltpu.VMEM((CHUNK,), jnp.float32),
            pltpu.VMEM((CHUNK,), jnp.float32),
            pltpu.VMEM((HISTW,), jnp.int32),
            pltpu.VMEM((STRIDE,), jnp.int32),
            pltpu.VMEM((STRIDE,), jnp.int32),
            pltpu.VMEM((16,), jnp.float32),
            pltpu.VMEM_SHARED((16, STRIDE), jnp.int32),
            pltpu.SemaphoreType.DMA,
            pltpu.SemaphoreType.DMA,
        ],
    )(_sc_select_body)
    return sel(luma_p2, luma_t2)


def _combine_body(sp_ref, st_ref, stats_ref, out_ref):
    inv_n = 1.0 / float(3 * 512 * 512)
    exposure = jnp.mean(jnp.abs(sp_ref[...] * inv_n - st_ref[...] * inv_n))
    st = stats_ref[...]
    d = jnp.abs(st[0:8, :] - st[8:16, :])
    lanemask = lax.broadcasted_iota(jnp.int32, (8, 16), 1) < 5
    hist = jnp.sum(jnp.where(lanemask, d, 0.0)) / 40.0
    out_ref[...] = jnp.full((1, 1), exposure + 0.5 * hist, jnp.float32)


def kernel(pred, target):
    luma_p, luma_t, sums_p, sums_t = pl.pallas_call(
        _luma_body,
        grid=(8,),
        in_specs=[
            pl.BlockSpec((1, 3, 512, 512), lambda i: (i, 0, 0, 0)),
            pl.BlockSpec((1, 3, 512, 512), lambda i: (i, 0, 0, 0)),
        ],
        out_specs=[
            pl.BlockSpec((1, 512, 512), lambda i: (i, 0, 0)),
            pl.BlockSpec((1, 512, 512), lambda i: (i, 0, 0)),
            pl.BlockSpec((1, 1, 128), lambda i: (i, 0, 0)),
            pl.BlockSpec((1, 1, 128), lambda i: (i, 0, 0)),
        ],
        out_shape=[
            jax.ShapeDtypeStruct((8, 512, 512), jnp.float32),
            jax.ShapeDtypeStruct((8, 512, 512), jnp.float32),
            jax.ShapeDtypeStruct((8, 1, 128), jnp.float32),
            jax.ShapeDtypeStruct((8, 1, 128), jnp.float32),
        ],
    )(pred, target)

    stats = _select_call(jnp.reshape(luma_p, (8, NELEM)),
                         jnp.reshape(luma_t, (8, NELEM)))

    out = pl.pallas_call(
        _combine_body,
        out_shape=jax.ShapeDtypeStruct((1, 1), jnp.float32),
    )(sums_p, sums_t, stats)
    return jnp.reshape(out, ())


# speedup vs baseline: 1.0674x; 1.0674x over previous
import functools

import jax
import jax.numpy as jnp
from jax import lax
from jax.experimental import pallas as pl
from jax.experimental.pallas import tpu as pltpu
from jax.experimental.pallas import tpu_sc as plsc

LANES = 16
NBIG = 2048
NSMALL = 256
STRIDE = 3 * NBIG
HISTW = LANES * STRIDE
NELEM = 512 * 512
HALFN = NELEM // 2
CHUNK = 4096
NCHUNK = HALFN // CHUNK
R25 = NELEM // 4 + 1
R50 = NELEM // 2 + 1
R75 = (3 * NELEM) // 4 + 1
UNROLL = 8


def _luma_body(p_ref, t_ref, lp_ref, lt_ref, sp_ref, st_ref):
    p0 = p_ref[0, 0]
    p1 = p_ref[0, 1]
    p2 = p_ref[0, 2]
    lp_ref[0] = 0.299 * p0 + 0.587 * p1 + 0.114 * p2
    sp = jnp.sum(p0) + jnp.sum(p1) + jnp.sum(p2)
    sp_ref[0, 0, :] = jnp.full((128,), sp, jnp.float32)
    t0 = t_ref[0, 0]
    t1 = t_ref[0, 1]
    t2 = t_ref[0, 2]
    lt_ref[0] = 0.299 * t0 + 0.587 * t1 + 0.114 * t2
    st = jnp.sum(t0) + jnp.sum(t1) + jnp.sum(t2)
    st_ref[0, 0, :] = jnp.full((128,), st, jnp.float32)


def _sc_select_body(lp_hbm, lt_hbm, out_hbm, buf0, buf1, hist, myflat, pflat,
                    vtmp, sh_hist, sem0, sem1):
    c = lax.axis_index("c")
    s = lax.axis_index("s")
    img = c * 8 + (s >> 1)
    half = s & 1
    part = s ^ 1
    lanes = lax.iota(jnp.int32, 16)
    ones = jnp.ones((16,), jnp.int32)
    zeros16 = jnp.zeros((16,), jnp.int32)
    lanebase = lanes * STRIDE
    base = half * HALFN

    def start_dma(ci, buf, sem):
        off = base + ci * CHUNK

        @pl.when(img < 8)
        def _():
            pltpu.async_copy(lp_hbm.at[img, pl.ds(off, CHUNK)], buf, sem)

        @pl.when(img >= 8)
        def _():
            pltpu.async_copy(lt_hbm.at[img - 8, pl.ds(off, CHUNK)], buf, sem)

    def wait_dma(buf, sem):
        pltpu.make_async_copy(lp_hbm.at[0, pl.ds(0, CHUNK)], buf, sem).wait()

    def stream_pass(per_vreg, car):
        car = jax.tree.map(jnp.asarray, car)
        def chunk_process(buf, c):
            def body(i, c2):
                x = buf[pl.ds(i, 16)]
                return per_vreg(x, c2)
            return plsc.parallel_loop(0, CHUNK, 16, unroll=UNROLL,
                                      carry=c)(body)

        def pair_body(i, c):
            ci0 = 2 * i
            wait_dma(buf0, sem0)
            start_dma(ci0 + 1, buf1, sem1)
            c = chunk_process(buf0, c)
            wait_dma(buf1, sem1)

            @pl.when(ci0 + 2 < NCHUNK)
            def _():
                start_dma(ci0 + 2, buf0, sem0)

            c = chunk_process(buf1, c)
            return c

        start_dma(0, buf0, sem0)
        return lax.fori_loop(0, NCHUNK // 2, pair_body, car)

    def zero_hist():
        def body(i):
            hist[pl.ds(i, 16)] = zeros16
        plsc.parallel_loop(0, HISTW, 16, unroll=8)(body)

    def zero_cols(ncols_pow2):
        sh = ncols_pow2.bit_length() - 1
        msk = ncols_pow2 - 1

        def body(i):
            hist[pl.ds(((i >> sh) * STRIDE) + (i & msk), 16)] = zeros16
        plsc.parallel_loop(0, LANES * ncols_pow2, 16, unroll=8)(body)

    def lane_reduce(ncols):
        def body(cv):
            acc = zeros16
            for l in range(LANES):
                acc = acc + hist[pl.ds(l * STRIDE + cv, 16)]
            myflat[pl.ds(cv, 16)] = acc
        plsc.parallel_loop(0, ncols, 16, unroll=2)(body)

    def publish(ncols):
        pltpu.sync_copy(myflat.at[pl.ds(0, ncols)],
                        sh_hist.at[s, pl.ds(0, ncols)])

    def fetch(ncols):
        pltpu.sync_copy(sh_hist.at[part, pl.ds(0, ncols)],
                        pflat.at[pl.ds(0, ncols)])

    def find(nbins, segoff, rank):
        def body(i, car):
            cum, b, cb = car
            o = segoff + i * 16
            v = myflat[pl.ds(o, 16)] + pflat[pl.ds(o, 16)]
            pc = plsc.cumsum(v) + cum
            lt = pc < rank
            b = b + jnp.sum(jnp.where(lt, ones, zeros16))
            cb = jnp.maximum(cb, jnp.max(jnp.where(lt, pc, zeros16)))
            return jnp.max(pc), b, cb
        z = jnp.int32(0)
        _, b, cb = lax.fori_loop(0, nbins // 16, body, (z, z, z))
        return b, cb

    zero_cols(NBIG)

    def p1_vreg(x, car2):
        vmin, vmax = car2
        bits = lax.bitcast_convert_type(x, jnp.int32)
        plsc.addupdate_scatter(hist, [lanebase + (bits >> 19)], ones)
        return jnp.minimum(vmin, x), jnp.maximum(vmax, x)

    vmin0 = jnp.full((16,), jnp.inf, jnp.float32)
    vmax0 = jnp.full((16,), -jnp.inf, jnp.float32)
    vmin, vmax = stream_pass(p1_vreg, (vmin0, vmax0))
    mn = jnp.min(vmin)
    mx = jnp.max(vmax)

    lane_reduce(NBIG)
    mmv = jnp.where(lanes == 1, mx, mn)
    myflat[pl.ds(NBIG, 16)] = lax.bitcast_convert_type(mmv, jnp.int32)
    publish(NBIG + 128)
    plsc.subcore_barrier()
    fetch(NBIG + 128)
    pmm = lax.bitcast_convert_type(pflat[pl.ds(NBIG, 16)], jnp.float32)
    mn = jnp.minimum(mn, jnp.min(pmm))
    mx = jnp.maximum(mx, jnp.max(pmm))
    plsc.subcore_barrier()

    b25, c25 = find(NBIG, 0, jnp.int32(R25))
    b50, c50 = find(NBIG, 0, jnp.int32(R50))
    b75, c75 = find(NBIG, 0, jnp.int32(R75))
    r2_25 = jnp.int32(R25) - c25
    r2_50 = jnp.int32(R50) - c50
    r2_75 = jnp.int32(R75) - c75

    zero_hist()

    def p2_vreg(x, car2):
        bits = lax.bitcast_convert_type(x, jnp.int32)
        top = bits >> 19
        addr = lanebase + ((bits >> 8) & (NBIG - 1))
        plsc.addupdate_scatter(hist, [addr], ones, mask=top == b25)
        plsc.addupdate_scatter(hist, [addr + NBIG], ones, mask=top == b50)
        plsc.addupdate_scatter(hist, [addr + 2 * NBIG], ones, mask=top == b75)
        return car2

    stream_pass(p2_vreg, 0)
    lane_reduce(3 * NBIG)
    publish(3 * NBIG)
    plsc.subcore_barrier()
    fetch(3 * NBIG)
    plsc.subcore_barrier()
    b2_25, c2_25 = find(NBIG, 0, r2_25)
    b2_50, c2_50 = find(NBIG, NBIG, r2_50)
    b2_75, c2_75 = find(NBIG, 2 * NBIG, r2_75)
    r3_25 = r2_25 - c2_25
    r3_50 = r2_50 - c2_50
    r3_75 = r2_75 - c2_75
    pre25 = (b25 << 11) | b2_25
    pre50 = (b50 << 11) | b2_50
    pre75 = (b75 << 11) | b2_75

    zero_cols(4 * NSMALL)

    def p3_vreg(x, car2):
        bits = lax.bitcast_convert_type(x, jnp.int32)
        hi = bits >> 8
        addr = lanebase + (bits & (NSMALL - 1))
        plsc.addupdate_scatter(hist, [addr], ones, mask=hi == pre25)
        plsc.addupdate_scatter(hist, [addr + NSMALL], ones, mask=hi == pre50)
        plsc.addupdate_scatter(hist, [addr + 2 * NSMALL], ones,
                               mask=hi == pre75)
        return car2

    stream_pass(p3_vreg, 0)
    lane_reduce(3 * NSMALL)
    publish(3 * NSMALL)
    plsc.subcore_barrier()
    fetch(3 * NSMALL)
    b3_25, _ = find(NSMALL, 0, r3_25)
    b3_50, _ = find(NSMALL, NSMALL, r3_50)
    b3_75, _ = find(NSMALL, 2 * NSMALL, r3_75)

    v25 = lax.bitcast_convert_type(
        jnp.broadcast_to((b25 << 19) | (b2_25 << 8) | b3_25, (16,)),
        jnp.float32)
    v50 = lax.bitcast_convert_type(
        jnp.broadcast_to((b50 << 19) | (b2_50 << 8) | b3_50, (16,)),
        jnp.float32)
    v75 = lax.bitcast_convert_type(
        jnp.broadcast_to((b75 << 19) | (b2_75 << 8) | b3_75, (16,)),
        jnp.float32)

    outv = jnp.where(lanes == 0, mn, jnp.zeros((16,), jnp.float32))
    outv = jnp.where(lanes == 1, v25, outv)
    outv = jnp.where(lanes == 2, v50, outv)
    outv = jnp.where(lanes == 3, v75, outv)
    outv = jnp.where(lanes == 4, mx, outv)
    vtmp[...] = outv

    @pl.when(half == 0)
    def _():
        pltpu.sync_copy(vtmp, out_hbm.at[img])


def _select_call(luma_p2, luma_t2):
    sel = functools.partial(
        pl.kernel,
        out_type=jax.ShapeDtypeStruct((16, 16), jnp.float32),
        mesh=plsc.VectorSubcoreMesh(core_axis_name="c", subcore_axis_name="s"),
        compiler_params=pltpu.CompilerParams(needs_layout_passes=False),
        scratch_types=[
            pltpu.VMEM((CHUNK,), jnp.float32),
            pltpu.VMEM((CHUNK,), jnp.float32),
            pltpu.VMEM((HISTW,), jnp.int32),
            pltpu.VMEM((STRIDE,), jnp.int32),
            pltpu.VMEM((STRIDE,), jnp.int32),
            pltpu.VMEM((16,), jnp.float32),
            pltpu.VMEM_SHARED((16, STRIDE), jnp.int32),
            pltpu.SemaphoreType.DMA,
            pltpu.SemaphoreType.DMA,
        ],
    )(_sc_select_body)
    return sel(luma_p2, luma_t2)


def _combine_body(sp_ref, st_ref, stats_ref, out_ref):
    inv_n = 1.0 / float(3 * 512 * 512)
    exposure = jnp.mean(jnp.abs(sp_ref[...] * inv_n - st_ref[...] * inv_n))
    st = stats_ref[...]
    d = jnp.abs(st[0:8, :] - st[8:16, :])
    lanemask = lax.broadcasted_iota(jnp.int32, (8, 16), 1) < 5
    hist = jnp.sum(jnp.where(lanemask, d, 0.0)) / 40.0
    out_ref[...] = jnp.full((1, 1), exposure + 0.5 * hist, jnp.float32)


def kernel(pred, target):
    luma_p, luma_t, sums_p, sums_t = pl.pallas_call(
        _luma_body,
        grid=(8,),
        in_specs=[
            pl.BlockSpec((1, 3, 512, 512), lambda i: (i, 0, 0, 0)),
            pl.BlockSpec((1, 3, 512, 512), lambda i: (i, 0, 0, 0)),
        ],
        out_specs=[
            pl.BlockSpec((1, 512, 512), lambda i: (i, 0, 0)),
            pl.BlockSpec((1, 512, 512), lambda i: (i, 0, 0)),
            pl.BlockSpec((1, 1, 128), lambda i: (i, 0, 0)),
            pl.BlockSpec((1, 1, 128), lambda i: (i, 0, 0)),
        ],
        out_shape=[
            jax.ShapeDtypeStruct((8, 512, 512), jnp.float32),
            jax.ShapeDtypeStruct((8, 512, 512), jnp.float32),
            jax.ShapeDtypeStruct((8, 1, 128), jnp.float32),
            jax.ShapeDtypeStruct((8, 1, 128), jnp.float32),
        ],
    )(pred, target)

    stats = _select_call(jnp.reshape(luma_p, (8, NELEM)),
                         jnp.reshape(luma_t, (8, NELEM)))

    out = pl.pallas_call(
        _combine_body,
        out_shape=jax.ShapeDtypeStruct((1, 1), jnp.float32),
    )(sums_p, sums_t, stats)
    return jnp.reshape(out, ())


# speedup vs baseline: 1.0684x; 1.0010x over previous
import functools

import jax
import jax.numpy as jnp
from jax import lax
from jax.experimental import pallas as pl
from jax.experimental.pallas import tpu as pltpu
from jax.experimental.pallas import tpu_sc as plsc

LANES = 16
NBIG = 2048
NSMALL = 256
STRIDE = 3 * NBIG
HISTW = LANES * STRIDE
NELEM = 512 * 512
HALFN = NELEM // 2
CHUNK = 4096
NCHUNK = HALFN // CHUNK
R25 = NELEM // 4 + 1
R50 = NELEM // 2 + 1
R75 = (3 * NELEM) // 4 + 1
UNROLL = 8


def _luma_body(p_ref, t_ref, lp_ref, lt_ref, sp_ref, st_ref):
    p0 = p_ref[0, 0]
    p1 = p_ref[0, 1]
    p2 = p_ref[0, 2]
    lp_ref[0] = 0.299 * p0 + 0.587 * p1 + 0.114 * p2
    sp = jnp.sum(p0) + jnp.sum(p1) + jnp.sum(p2)
    sp_ref[0, 0, :] = jnp.full((128,), sp, jnp.float32)
    t0 = t_ref[0, 0]
    t1 = t_ref[0, 1]
    t2 = t_ref[0, 2]
    lt_ref[0] = 0.299 * t0 + 0.587 * t1 + 0.114 * t2
    st = jnp.sum(t0) + jnp.sum(t1) + jnp.sum(t2)
    st_ref[0, 0, :] = jnp.full((128,), st, jnp.float32)


def _sc_select_body(lp_hbm, lt_hbm, out_hbm, buf0, buf1, hist, myflat, pflat,
                    vtmp, sh_hist, sem0, sem1):
    c = lax.axis_index("c")
    s = lax.axis_index("s")
    img = c * 8 + (s >> 1)
    half = s & 1
    part = s ^ 1
    lanes = lax.iota(jnp.int32, 16)
    ones = jnp.ones((16,), jnp.int32)
    zeros16 = jnp.zeros((16,), jnp.int32)
    lanebase = lanes * STRIDE
    base = half * HALFN

    def start_dma(ci, buf, sem):
        off = base + ci * CHUNK

        @pl.when(img < 8)
        def _():
            pltpu.async_copy(lp_hbm.at[img, pl.ds(off, CHUNK)], buf, sem)

        @pl.when(img >= 8)
        def _():
            pltpu.async_copy(lt_hbm.at[img - 8, pl.ds(off, CHUNK)], buf, sem)

    def wait_dma(buf, sem):
        pltpu.make_async_copy(lp_hbm.at[0, pl.ds(0, CHUNK)], buf, sem).wait()

    def stream_pass(per_vreg, car):
        car = jax.tree.map(jnp.asarray, car)
        def chunk_process(buf, c):
            def body(i, c2):
                x = buf[pl.ds(i, 16)]
                return per_vreg(x, c2)
            return plsc.parallel_loop(0, CHUNK, 16, unroll=UNROLL,
                                      carry=c)(body)

        def pair_body(i, c):
            ci0 = 2 * i
            wait_dma(buf0, sem0)
            start_dma(ci0 + 1, buf1, sem1)
            c = chunk_process(buf0, c)
            wait_dma(buf1, sem1)

            @pl.when(ci0 + 2 < NCHUNK)
            def _():
                start_dma(ci0 + 2, buf0, sem0)

            c = chunk_process(buf1, c)
            return c

        start_dma(0, buf0, sem0)
        return lax.fori_loop(0, NCHUNK // 2, pair_body, car)

    def zero_hist():
        def body(i):
            hist[pl.ds(i, 16)] = zeros16
        plsc.parallel_loop(0, HISTW, 16, unroll=8)(body)

    def zero_cols(ncols_pow2):
        sh = ncols_pow2.bit_length() - 1
        msk = ncols_pow2 - 1

        def body(i):
            hist[pl.ds(((i >> sh) * STRIDE) + (i & msk), 16)] = zeros16
        plsc.parallel_loop(0, LANES * ncols_pow2, 16, unroll=8)(body)

    def lane_reduce(ncols):
        def body(cv):
            acc = zeros16
            for l in range(LANES):
                acc = acc + hist[pl.ds(l * STRIDE + cv, 16)]
            myflat[pl.ds(cv, 16)] = acc
        plsc.parallel_loop(0, ncols, 16, unroll=2)(body)

    def publish(ncols):
        pltpu.sync_copy(myflat.at[pl.ds(0, ncols)],
                        sh_hist.at[s, pl.ds(0, ncols)])

    def fetch(ncols):
        pltpu.sync_copy(sh_hist.at[part, pl.ds(0, ncols)],
                        pflat.at[pl.ds(0, ncols)])

    def find(nbins, segoff, rank):
        def body(i, car):
            cum, b, cb = car
            o = segoff + i * 16
            v = myflat[pl.ds(o, 16)] + pflat[pl.ds(o, 16)]
            pc = plsc.cumsum(v) + cum
            lt = pc < rank
            b = b + jnp.sum(jnp.where(lt, ones, zeros16))
            cb = jnp.maximum(cb, jnp.max(jnp.where(lt, pc, zeros16)))
            return jnp.max(pc), b, cb
        z = jnp.int32(0)
        _, b, cb = lax.fori_loop(0, nbins // 16, body, (z, z, z))
        return b, cb

    scope = jax.named_scope
    with scope("z1"):
        zero_cols(NBIG)

    def p1_vreg(x, car2):
        vmin, vmax = car2
        bits = lax.bitcast_convert_type(x, jnp.int32)
        plsc.addupdate_scatter(hist, [lanebase + (bits >> 19)], ones)
        return jnp.minimum(vmin, x), jnp.maximum(vmax, x)

    vmin0 = jnp.full((16,), jnp.inf, jnp.float32)
    vmax0 = jnp.full((16,), -jnp.inf, jnp.float32)
    with scope("p1"):
        vmin, vmax = stream_pass(p1_vreg, (vmin0, vmax0))
    mn = jnp.min(vmin)
    mx = jnp.max(vmax)

    with scope("m1"):
        lane_reduce(NBIG)
    mmv = jnp.where(lanes == 1, mx, mn)
    myflat[pl.ds(NBIG, 16)] = lax.bitcast_convert_type(mmv, jnp.int32)
    publish(NBIG + 128)
    plsc.subcore_barrier()
    fetch(NBIG + 128)
    pmm = lax.bitcast_convert_type(pflat[pl.ds(NBIG, 16)], jnp.float32)
    mn = jnp.minimum(mn, jnp.min(pmm))
    mx = jnp.maximum(mx, jnp.max(pmm))
    plsc.subcore_barrier()

    with scope("f1"):
        b25, c25 = find(NBIG, 0, jnp.int32(R25))
        b50, c50 = find(NBIG, 0, jnp.int32(R50))
        b75, c75 = find(NBIG, 0, jnp.int32(R75))
    r2_25 = jnp.int32(R25) - c25
    r2_50 = jnp.int32(R50) - c50
    r2_75 = jnp.int32(R75) - c75

    with scope("z2"):
        zero_hist()

    def p2_vreg(x, car2):
        bits = lax.bitcast_convert_type(x, jnp.int32)
        top = bits >> 19
        addr = lanebase + ((bits >> 8) & (NBIG - 1))
        plsc.addupdate_scatter(hist, [addr], ones, mask=top == b25)
        plsc.addupdate_scatter(hist, [addr + NBIG], ones, mask=top == b50)
        plsc.addupdate_scatter(hist, [addr + 2 * NBIG], ones, mask=top == b75)
        return car2

    with scope("p2"):
        stream_pass(p2_vreg, 0)
    with scope("m2"):
        lane_reduce(3 * NBIG)
    publish(3 * NBIG)
    plsc.subcore_barrier()
    fetch(3 * NBIG)
    plsc.subcore_barrier()
    b2_25, c2_25 = find(NBIG, 0, r2_25)
    b2_50, c2_50 = find(NBIG, NBIG, r2_50)
    b2_75, c2_75 = find(NBIG, 2 * NBIG, r2_75)
    r3_25 = r2_25 - c2_25
    r3_50 = r2_50 - c2_50
    r3_75 = r2_75 - c2_75
    pre25 = (b25 << 11) | b2_25
    pre50 = (b50 << 11) | b2_50
    pre75 = (b75 << 11) | b2_75

    with scope("z3"):
        zero_cols(4 * NSMALL)

    def p3_vreg(x, car2):
        bits = lax.bitcast_convert_type(x, jnp.int32)
        hi = bits >> 8
        addr = lanebase + (bits & (NSMALL - 1))
        plsc.addupdate_scatter(hist, [addr], ones, mask=hi == pre25)
        plsc.addupdate_scatter(hist, [addr + NSMALL], ones, mask=hi == pre50)
        plsc.addupdate_scatter(hist, [addr + 2 * NSMALL], ones,
                               mask=hi == pre75)
        return car2

    with scope("p3"):
        stream_pass(p3_vreg, 0)
    with scope("m3"):
        lane_reduce(3 * NSMALL)
    publish(3 * NSMALL)
    plsc.subcore_barrier()
    fetch(3 * NSMALL)
    b3_25, _ = find(NSMALL, 0, r3_25)
    b3_50, _ = find(NSMALL, NSMALL, r3_50)
    b3_75, _ = find(NSMALL, 2 * NSMALL, r3_75)

    v25 = lax.bitcast_convert_type(
        jnp.broadcast_to((b25 << 19) | (b2_25 << 8) | b3_25, (16,)),
        jnp.float32)
    v50 = lax.bitcast_convert_type(
        jnp.broadcast_to((b50 << 19) | (b2_50 << 8) | b3_50, (16,)),
        jnp.float32)
    v75 = lax.bitcast_convert_type(
        jnp.broadcast_to((b75 << 19) | (b2_75 << 8) | b3_75, (16,)),
        jnp.float32)

    outv = jnp.where(lanes == 0, mn, jnp.zeros((16,), jnp.float32))
    outv = jnp.where(lanes == 1, v25, outv)
    outv = jnp.where(lanes == 2, v50, outv)
    outv = jnp.where(lanes == 3, v75, outv)
    outv = jnp.where(lanes == 4, mx, outv)
    vtmp[...] = outv

    @pl.when(half == 0)
    def _():
        pltpu.sync_copy(vtmp, out_hbm.at[img])


def _select_call(luma_p2, luma_t2):
    sel = functools.partial(
        pl.kernel,
        out_type=jax.ShapeDtypeStruct((16, 16), jnp.float32),
        mesh=plsc.VectorSubcoreMesh(core_axis_name="c", subcore_axis_name="s"),
        compiler_params=pltpu.CompilerParams(needs_layout_passes=False),
        scratch_types=[
            pltpu.VMEM((CHUNK,), jnp.float32),
            pltpu.VMEM((CHUNK,), jnp.float32),
            pltpu.VMEM((HISTW,), jnp.int32),
            pltpu.VMEM((STRIDE,), jnp.int32),
            pltpu.VMEM((STRIDE,), jnp.int32),
            pltpu.VMEM((16,), jnp.float32),
            pltpu.VMEM_SHARED((16, STRIDE), jnp.int32),
            pltpu.SemaphoreType.DMA,
            pltpu.SemaphoreType.DMA,
        ],
    )(_sc_select_body)
    return sel(luma_p2, luma_t2)


def _combine_body(sp_ref, st_ref, stats_ref, out_ref):
    inv_n = 1.0 / float(3 * 512 * 512)
    exposure = jnp.mean(jnp.abs(sp_ref[...] * inv_n - st_ref[...] * inv_n))
    st = stats_ref[...]
    d = jnp.abs(st[0:8, :] - st[8:16, :])
    lanemask = lax.broadcasted_iota(jnp.int32, (8, 16), 1) < 5
    hist = jnp.sum(jnp.where(lanemask, d, 0.0)) / 40.0
    out_ref[...] = jnp.full((1, 1), exposure + 0.5 * hist, jnp.float32)


def kernel(pred, target):
    luma_p, luma_t, sums_p, sums_t = pl.pallas_call(
        _luma_body,
        grid=(8,),
        in_specs=[
            pl.BlockSpec((1, 3, 512, 512), lambda i: (i, 0, 0, 0)),
            pl.BlockSpec((1, 3, 512, 512), lambda i: (i, 0, 0, 0)),
        ],
        out_specs=[
            pl.BlockSpec((1, 512, 512), lambda i: (i, 0, 0)),
            pl.BlockSpec((1, 512, 512), lambda i: (i, 0, 0)),
            pl.BlockSpec((1, 1, 128), lambda i: (i, 0, 0)),
            pl.BlockSpec((1, 1, 128), lambda i: (i, 0, 0)),
        ],
        out_shape=[
            jax.ShapeDtypeStruct((8, 512, 512), jnp.float32),
            jax.ShapeDtypeStruct((8, 512, 512), jnp.float32),
            jax.ShapeDtypeStruct((8, 1, 128), jnp.float32),
            jax.ShapeDtypeStruct((8, 1, 128), jnp.float32),
        ],
    )(pred, target)

    stats = _select_call(jnp.reshape(luma_p, (8, NELEM)),
                         jnp.reshape(luma_t, (8, NELEM)))

    out = pl.pallas_call(
        _combine_body,
        out_shape=jax.ShapeDtypeStruct((1, 1), jnp.float32),
    )(sums_p, sums_t, stats)
    return jnp.reshape(out, ())


# speedup vs baseline: 1.1572x; 1.0831x over previous
import functools

import jax
import jax.numpy as jnp
from jax import lax
from jax.experimental import pallas as pl
from jax.experimental.pallas import tpu as pltpu
from jax.experimental.pallas import tpu_sc as plsc

LANES = 16
NBIG = 2048
NSMALL = 256
STRIDE = 3 * NBIG
HISTW = LANES * STRIDE
NELEM = 512 * 512
HALFN = NELEM // 2
CHUNK = 8192
NCHUNK = HALFN // CHUNK
R25 = NELEM // 4 + 1
R50 = NELEM // 2 + 1
R75 = (3 * NELEM) // 4 + 1
UNROLL = 16


def _luma_body(p_ref, t_ref, lp_ref, lt_ref, sp_ref, st_ref):
    p0 = p_ref[0, 0]
    p1 = p_ref[0, 1]
    p2 = p_ref[0, 2]
    lp_ref[0] = 0.299 * p0 + 0.587 * p1 + 0.114 * p2
    sp = jnp.sum(p0) + jnp.sum(p1) + jnp.sum(p2)
    sp_ref[0, 0, :] = jnp.full((128,), sp, jnp.float32)
    t0 = t_ref[0, 0]
    t1 = t_ref[0, 1]
    t2 = t_ref[0, 2]
    lt_ref[0] = 0.299 * t0 + 0.587 * t1 + 0.114 * t2
    st = jnp.sum(t0) + jnp.sum(t1) + jnp.sum(t2)
    st_ref[0, 0, :] = jnp.full((128,), st, jnp.float32)


def _sc_select_body(lp_hbm, lt_hbm, out_hbm, buf0, buf1, hist, myflat,
                    vtmp, sh_hist, sem0, sem1):
    c = lax.axis_index("c")
    s = lax.axis_index("s")
    img = c * 8 + (s >> 1)
    half = s & 1
    part = s ^ 1
    lanes = lax.iota(jnp.int32, 16)
    ones = jnp.ones((16,), jnp.int32)
    zeros16 = jnp.zeros((16,), jnp.int32)
    lanebase = lanes * STRIDE
    base = half * HALFN

    def start_dma(ci, buf, sem):
        off = base + ci * CHUNK

        @pl.when(img < 8)
        def _():
            pltpu.async_copy(lp_hbm.at[img, pl.ds(off, CHUNK)], buf, sem)

        @pl.when(img >= 8)
        def _():
            pltpu.async_copy(lt_hbm.at[img - 8, pl.ds(off, CHUNK)], buf, sem)

    def wait_dma(buf, sem):
        pltpu.make_async_copy(lp_hbm.at[0, pl.ds(0, CHUNK)], buf, sem).wait()

    def stream_pass(per_vreg, car):
        car = jax.tree.map(jnp.asarray, car)
        def chunk_process(buf, c):
            def body(i, c2):
                x = buf[pl.ds(i, 16)]
                return per_vreg(x, c2)
            return plsc.parallel_loop(0, CHUNK, 16, unroll=UNROLL,
                                      carry=c)(body)

        def pair_body(i, c):
            ci0 = 2 * i
            wait_dma(buf0, sem0)
            start_dma(ci0 + 1, buf1, sem1)
            c = chunk_process(buf0, c)
            wait_dma(buf1, sem1)

            @pl.when(ci0 + 2 < NCHUNK)
            def _():
                start_dma(ci0 + 2, buf0, sem0)

            c = chunk_process(buf1, c)
            return c

        start_dma(0, buf0, sem0)
        return lax.fori_loop(0, NCHUNK // 2, pair_body, car)

    def zero_hist():
        def body(i):
            hist[pl.ds(i, 16)] = zeros16
        plsc.parallel_loop(0, HISTW, 16, unroll=8)(body)

    def zero_cols(ncols_pow2):
        sh = ncols_pow2.bit_length() - 1
        msk = ncols_pow2 - 1

        def body(i):
            hist[pl.ds(((i >> sh) * STRIDE) + (i & msk), 16)] = zeros16
        plsc.parallel_loop(0, LANES * ncols_pow2, 16, unroll=8)(body)

    def lane_reduce(ncols):
        def body(cv):
            acc = zeros16
            for l in range(LANES):
                acc = acc + hist[pl.ds(l * STRIDE + cv, 16)]
            myflat[pl.ds(cv, 16)] = acc
        plsc.parallel_loop(0, ncols, 16, unroll=2)(body)

    def publish(ncols):
        pltpu.sync_copy(myflat.at[pl.ds(0, ncols)],
                        sh_hist.at[s, pl.ds(0, ncols)])

    def fetch(ncols):
        pltpu.sync_copy(sh_hist.at[part, pl.ds(0, ncols)],
                        hist.at[pl.ds(0, ncols)])

    def find_shared(nbins, segoff, r1, r2, r3):
        def body(i, car):
            cum, st = car
            o = segoff + i * 16
            v = myflat[pl.ds(o, 16)] + hist[pl.ds(o, 16)]
            pc = plsc.cumsum(v) + cum
            new = []
            for (b, cb), r in zip(st, (r1, r2, r3)):
                lt = pc < r
                b = b + jnp.sum(jnp.where(lt, ones, zeros16))
                cb = jnp.maximum(cb, jnp.max(jnp.where(lt, pc, zeros16)))
                new.append((b, cb))
            return jnp.max(pc), tuple(new)
        z = jnp.int32(0)
        _, st = lax.fori_loop(0, nbins // 16, body,
                              (z, ((z, z), (z, z), (z, z))))
        return st

    def find_multi(nbins, segoffs, ranks):
        def body(i, car):
            new = []
            for (cum, b, cb), o0, r in zip(car, segoffs, ranks):
                o = o0 + i * 16
                v = myflat[pl.ds(o, 16)] + hist[pl.ds(o, 16)]
                pc = plsc.cumsum(v) + cum
                lt = pc < r
                b = b + jnp.sum(jnp.where(lt, ones, zeros16))
                cb = jnp.maximum(cb, jnp.max(jnp.where(lt, pc, zeros16)))
                new.append((jnp.max(pc), b, cb))
            return tuple(new)
        z = jnp.int32(0)
        st = lax.fori_loop(0, nbins // 16, body, ((z,) * 3,) * 3)
        return tuple((b, cb) for _, b, cb in st)

    scope = jax.named_scope
    with scope("z1"):
        zero_cols(NBIG)

    def p1_vreg(x, car2):
        vmin, vmax = car2
        bits = lax.bitcast_convert_type(x, jnp.int32)
        plsc.addupdate_scatter(hist, [lanebase + (bits >> 19)], ones)
        return jnp.minimum(vmin, x), jnp.maximum(vmax, x)

    vmin0 = jnp.full((16,), jnp.inf, jnp.float32)
    vmax0 = jnp.full((16,), -jnp.inf, jnp.float32)
    with scope("p1"):
        vmin, vmax = stream_pass(p1_vreg, (vmin0, vmax0))
    mn = jnp.min(vmin)
    mx = jnp.max(vmax)

    with scope("m1"):
        lane_reduce(NBIG)
    mmv = jnp.where(lanes == 1, mx, mn)
    myflat[pl.ds(NBIG, 16)] = lax.bitcast_convert_type(mmv, jnp.int32)
    publish(NBIG + 128)
    plsc.subcore_barrier()
    fetch(NBIG + 128)
    pmm = lax.bitcast_convert_type(hist[pl.ds(NBIG, 16)], jnp.float32)
    mn = jnp.minimum(mn, jnp.min(pmm))
    mx = jnp.maximum(mx, jnp.max(pmm))
    plsc.subcore_barrier()

    with scope("f1"):
        ((b25, c25), (b50, c50), (b75, c75)) = find_shared(
            NBIG, 0, jnp.int32(R25), jnp.int32(R50), jnp.int32(R75))
    r2_25 = jnp.int32(R25) - c25
    r2_50 = jnp.int32(R50) - c50
    r2_75 = jnp.int32(R75) - c75

    with scope("z2"):
        zero_hist()

    def p2_vreg(x, car2):
        bits = lax.bitcast_convert_type(x, jnp.int32)
        top = bits >> 19
        addr = lanebase + ((bits >> 8) & (NBIG - 1))
        plsc.addupdate_scatter(hist, [addr], ones, mask=top == b25)
        plsc.addupdate_scatter(hist, [addr + NBIG], ones, mask=top == b50)
        plsc.addupdate_scatter(hist, [addr + 2 * NBIG], ones, mask=top == b75)
        return car2

    with scope("p2"):
        stream_pass(p2_vreg, 0)
    with scope("m2"):
        lane_reduce(3 * NBIG)
    publish(3 * NBIG)
    plsc.subcore_barrier()
    fetch(3 * NBIG)
    plsc.subcore_barrier()
    ((b2_25, c2_25), (b2_50, c2_50), (b2_75, c2_75)) = find_multi(
        NBIG, (0, NBIG, 2 * NBIG), (r2_25, r2_50, r2_75))
    r3_25 = r2_25 - c2_25
    r3_50 = r2_50 - c2_50
    r3_75 = r2_75 - c2_75
    pre25 = (b25 << 11) | b2_25
    pre50 = (b50 << 11) | b2_50
    pre75 = (b75 << 11) | b2_75

    with scope("z3"):
        zero_cols(4 * NSMALL)

    def p3_vreg(x, car2):
        bits = lax.bitcast_convert_type(x, jnp.int32)
        hi = bits >> 8
        addr = lanebase + (bits & (NSMALL - 1))
        plsc.addupdate_scatter(hist, [addr], ones, mask=hi == pre25)
        plsc.addupdate_scatter(hist, [addr + NSMALL], ones, mask=hi == pre50)
        plsc.addupdate_scatter(hist, [addr + 2 * NSMALL], ones,
                               mask=hi == pre75)
        return car2

    with scope("p3"):
        stream_pass(p3_vreg, 0)
    with scope("m3"):
        lane_reduce(3 * NSMALL)
    publish(3 * NSMALL)
    plsc.subcore_barrier()
    fetch(3 * NSMALL)
    ((b3_25, _), (b3_50, _), (b3_75, _)) = find_multi(
        NSMALL, (0, NSMALL, 2 * NSMALL), (r3_25, r3_50, r3_75))

    v25 = lax.bitcast_convert_type(
        jnp.broadcast_to((b25 << 19) | (b2_25 << 8) | b3_25, (16,)),
        jnp.float32)
    v50 = lax.bitcast_convert_type(
        jnp.broadcast_to((b50 << 19) | (b2_50 << 8) | b3_50, (16,)),
        jnp.float32)
    v75 = lax.bitcast_convert_type(
        jnp.broadcast_to((b75 << 19) | (b2_75 << 8) | b3_75, (16,)),
        jnp.float32)

    outv = jnp.where(lanes == 0, mn, jnp.zeros((16,), jnp.float32))
    outv = jnp.where(lanes == 1, v25, outv)
    outv = jnp.where(lanes == 2, v50, outv)
    outv = jnp.where(lanes == 3, v75, outv)
    outv = jnp.where(lanes == 4, mx, outv)
    vtmp[...] = outv

    @pl.when(half == 0)
    def _():
        pltpu.sync_copy(vtmp, out_hbm.at[img])


def _select_call(luma_p2, luma_t2):
    sel = functools.partial(
        pl.kernel,
        out_type=jax.ShapeDtypeStruct((16, 16), jnp.float32),
        mesh=plsc.VectorSubcoreMesh(core_axis_name="c", subcore_axis_name="s"),
        compiler_params=pltpu.CompilerParams(needs_layout_passes=False),
        scratch_types=[
            pltpu.VMEM((CHUNK,), jnp.float32),
            pltpu.VMEM((CHUNK,), jnp.float32),
            pltpu.VMEM((HISTW,), jnp.int32),
            pltpu.VMEM((STRIDE,), jnp.int32),
            pltpu.VMEM((16,), jnp.float32),
            pltpu.VMEM_SHARED((16, STRIDE), jnp.int32),
            pltpu.SemaphoreType.DMA,
            pltpu.SemaphoreType.DMA,
        ],
    )(_sc_select_body)
    return sel(luma_p2, luma_t2)


def _combine_body(sp_ref, st_ref, stats_ref, out_ref):
    inv_n = 1.0 / float(3 * 512 * 512)
    exposure = jnp.mean(jnp.abs(sp_ref[...] * inv_n - st_ref[...] * inv_n))
    st = stats_ref[...]
    d = jnp.abs(st[0:8, :] - st[8:16, :])
    lanemask = lax.broadcasted_iota(jnp.int32, (8, 16), 1) < 5
    hist = jnp.sum(jnp.where(lanemask, d, 0.0)) / 40.0
    out_ref[...] = jnp.full((1, 1), exposure + 0.5 * hist, jnp.float32)


def kernel(pred, target):
    luma_p, luma_t, sums_p, sums_t = pl.pallas_call(
        _luma_body,
        grid=(8,),
        in_specs=[
            pl.BlockSpec((1, 3, 512, 512), lambda i: (i, 0, 0, 0)),
            pl.BlockSpec((1, 3, 512, 512), lambda i: (i, 0, 0, 0)),
        ],
        out_specs=[
            pl.BlockSpec((1, 512, 512), lambda i: (i, 0, 0)),
            pl.BlockSpec((1, 512, 512), lambda i: (i, 0, 0)),
            pl.BlockSpec((1, 1, 128), lambda i: (i, 0, 0)),
            pl.BlockSpec((1, 1, 128), lambda i: (i, 0, 0)),
        ],
        out_shape=[
            jax.ShapeDtypeStruct((8, 512, 512), jnp.float32),
            jax.ShapeDtypeStruct((8, 512, 512), jnp.float32),
            jax.ShapeDtypeStruct((8, 1, 128), jnp.float32),
            jax.ShapeDtypeStruct((8, 1, 128), jnp.float32),
        ],
    )(pred, target)

    stats = _select_call(jnp.reshape(luma_p, (8, NELEM)),
                         jnp.reshape(luma_t, (8, NELEM)))

    out = pl.pallas_call(
        _combine_body,
        out_shape=jax.ShapeDtypeStruct((1, 1), jnp.float32),
    )(sums_p, sums_t, stats)
    return jnp.reshape(out, ())


# speedup vs baseline: 1.2036x; 1.0401x over previous
import functools

import jax
import jax.numpy as jnp
from jax import lax
from jax.experimental import pallas as pl
from jax.experimental.pallas import tpu as pltpu
from jax.experimental.pallas import tpu_sc as plsc

LANES = 16
NBIG = 2048
NSMALL = 256
STRIDE = 3 * NBIG
HISTW = LANES * STRIDE
NELEM = 512 * 512
HALFN = NELEM // 2
CHUNK = 8192
NCHUNK = HALFN // CHUNK
R25 = NELEM // 4 + 1
R50 = NELEM // 2 + 1
R75 = (3 * NELEM) // 4 + 1
UNROLL = 16


def _luma_body(p_ref, t_ref, lp_ref, lt_ref, sp_ref, st_ref):
    p0 = p_ref[0, 0]
    p1 = p_ref[0, 1]
    p2 = p_ref[0, 2]
    lp_ref[0] = 0.299 * p0 + 0.587 * p1 + 0.114 * p2
    sp = jnp.sum(p0) + jnp.sum(p1) + jnp.sum(p2)
    sp_ref[0, 0, :] = jnp.full((128,), sp, jnp.float32)
    t0 = t_ref[0, 0]
    t1 = t_ref[0, 1]
    t2 = t_ref[0, 2]
    lt_ref[0] = 0.299 * t0 + 0.587 * t1 + 0.114 * t2
    st = jnp.sum(t0) + jnp.sum(t1) + jnp.sum(t2)
    st_ref[0, 0, :] = jnp.full((128,), st, jnp.float32)


def _sc_select_body(lp_hbm, lt_hbm, out_hbm, buf0, buf1, hist, myflat,
                    vtmp, sh_hist, sem0, sem1):
    c = lax.axis_index("c")
    s = lax.axis_index("s")
    img = c * 8 + (s >> 1)
    half = s & 1
    part = s ^ 1
    lanes = lax.iota(jnp.int32, 16)
    ones = jnp.ones((16,), jnp.int32)
    zeros16 = jnp.zeros((16,), jnp.int32)
    lanebase = lanes * STRIDE
    base = half * HALFN

    def start_dma(ci, buf, sem):
        off = base + ci * CHUNK

        @pl.when(img < 8)
        def _():
            pltpu.async_copy(lp_hbm.at[img, pl.ds(off, CHUNK)], buf, sem)

        @pl.when(img >= 8)
        def _():
            pltpu.async_copy(lt_hbm.at[img - 8, pl.ds(off, CHUNK)], buf, sem)

    def wait_dma(buf, sem):
        pltpu.make_async_copy(lp_hbm.at[0, pl.ds(0, CHUNK)], buf, sem).wait()

    def stream_pass(per_vreg, car, unroll=UNROLL):
        car = jax.tree.map(jnp.asarray, car)
        def chunk_process(buf, c):
            def body(i, c2):
                x = buf[pl.ds(i, 16)]
                return per_vreg(x, c2)
            return plsc.parallel_loop(0, CHUNK, 16, unroll=unroll,
                                      carry=c)(body)

        def pair_body(i, c):
            ci0 = 2 * i
            wait_dma(buf0, sem0)
            start_dma(ci0 + 1, buf1, sem1)
            c = chunk_process(buf0, c)
            wait_dma(buf1, sem1)

            @pl.when(ci0 + 2 < NCHUNK)
            def _():
                start_dma(ci0 + 2, buf0, sem0)

            c = chunk_process(buf1, c)
            return c

        start_dma(0, buf0, sem0)
        return lax.fori_loop(0, NCHUNK // 2, pair_body, car)

    def zero_hist():
        def body(i):
            hist[pl.ds(i, 16)] = zeros16
        plsc.parallel_loop(0, HISTW, 16, unroll=8)(body)

    def zero_cols(ncols_pow2):
        sh = ncols_pow2.bit_length() - 1
        msk = ncols_pow2 - 1

        def body(i):
            hist[pl.ds(((i >> sh) * STRIDE) + (i & msk), 16)] = zeros16
        plsc.parallel_loop(0, LANES * ncols_pow2, 16, unroll=8)(body)

    def lane_reduce(ncols):
        def body(cv):
            acc = zeros16
            for l in range(LANES):
                acc = acc + hist[pl.ds(l * STRIDE + cv, 16)]
            myflat[pl.ds(cv, 16)] = acc
        plsc.parallel_loop(0, ncols, 16, unroll=2)(body)

    def publish(ncols):
        pltpu.sync_copy(myflat.at[pl.ds(0, ncols)],
                        sh_hist.at[s, pl.ds(0, ncols)])

    def fetch(ncols):
        pltpu.sync_copy(sh_hist.at[part, pl.ds(0, ncols)],
                        hist.at[pl.ds(0, ncols)])

    def find_shared(nbins, segoff, r1, r2, r3):
        def body(i, car):
            cum, st = car
            o = segoff + i * 16
            v = myflat[pl.ds(o, 16)] + hist[pl.ds(o, 16)]
            pc = plsc.cumsum(v) + cum
            new = []
            for (b, cb), r in zip(st, (r1, r2, r3)):
                lt = pc < r
                b = b + jnp.sum(jnp.where(lt, ones, zeros16))
                cb = jnp.maximum(cb, jnp.max(jnp.where(lt, pc, zeros16)))
                new.append((b, cb))
            return jnp.max(pc), tuple(new)
        z = jnp.int32(0)
        _, st = lax.fori_loop(0, nbins // 16, body,
                              (z, ((z, z), (z, z), (z, z))))
        return st

    def find_multi(nbins, segoffs, ranks):
        def body(i, car):
            new = []
            for (cum, b, cb), o0, r in zip(car, segoffs, ranks):
                o = o0 + i * 16
                v = myflat[pl.ds(o, 16)] + hist[pl.ds(o, 16)]
                pc = plsc.cumsum(v) + cum
                lt = pc < r
                b = b + jnp.sum(jnp.where(lt, ones, zeros16))
                cb = jnp.maximum(cb, jnp.max(jnp.where(lt, pc, zeros16)))
                new.append((jnp.max(pc), b, cb))
            return tuple(new)
        z = jnp.int32(0)
        st = lax.fori_loop(0, nbins // 16, body, ((z,) * 3,) * 3)
        return tuple((b, cb) for _, b, cb in st)

    scope = jax.named_scope
    with scope("z1"):
        zero_cols(NBIG)

    def p1_vreg(x, car2):
        vmin, vmax = car2
        bits = lax.bitcast_convert_type(x, jnp.int32)
        plsc.addupdate_scatter(hist, [lanebase + (bits >> 19)], ones)
        return jnp.minimum(vmin, x), jnp.maximum(vmax, x)

    vmin0 = jnp.full((16,), jnp.inf, jnp.float32)
    vmax0 = jnp.full((16,), -jnp.inf, jnp.float32)
    with scope("p1"):
        vmin, vmax = stream_pass(p1_vreg, (vmin0, vmax0))
    mn = jnp.min(vmin)
    mx = jnp.max(vmax)

    with scope("m1"):
        lane_reduce(NBIG)
    mmv = jnp.where(lanes == 1, mx, mn)
    myflat[pl.ds(NBIG, 16)] = lax.bitcast_convert_type(mmv, jnp.int32)
    publish(NBIG + 128)
    plsc.subcore_barrier()
    fetch(NBIG + 128)
    pmm = lax.bitcast_convert_type(hist[pl.ds(NBIG, 16)], jnp.float32)
    mn = jnp.minimum(mn, jnp.min(pmm))
    mx = jnp.maximum(mx, jnp.max(pmm))
    plsc.subcore_barrier()

    with scope("f1"):
        ((b25, c25), (b50, c50), (b75, c75)) = find_shared(
            NBIG, 0, jnp.int32(R25), jnp.int32(R50), jnp.int32(R75))
    r2_25 = jnp.int32(R25) - c25
    r2_50 = jnp.int32(R50) - c50
    r2_75 = jnp.int32(R75) - c75

    with scope("z2"):
        zero_hist()

    def p2_vreg(x, car2):
        bits = lax.bitcast_convert_type(x, jnp.int32)
        top = bits >> 19
        addr = lanebase + ((bits >> 8) & (NBIG - 1))
        plsc.addupdate_scatter(hist, [addr], ones, mask=top == b25)
        plsc.addupdate_scatter(hist, [addr + NBIG], ones, mask=top == b50)
        plsc.addupdate_scatter(hist, [addr + 2 * NBIG], ones, mask=top == b75)
        return car2

    with scope("p2"):
        stream_pass(p2_vreg, 0, unroll=8)
    with scope("m2"):
        lane_reduce(3 * NBIG)
    publish(3 * NBIG)
    plsc.subcore_barrier()
    fetch(3 * NBIG)
    plsc.subcore_barrier()
    ((b2_25, c2_25), (b2_50, c2_50), (b2_75, c2_75)) = find_multi(
        NBIG, (0, NBIG, 2 * NBIG), (r2_25, r2_50, r2_75))
    r3_25 = r2_25 - c2_25
    r3_50 = r2_50 - c2_50
    r3_75 = r2_75 - c2_75
    pre25 = (b25 << 11) | b2_25
    pre50 = (b50 << 11) | b2_50
    pre75 = (b75 << 11) | b2_75

    with scope("z3"):
        zero_cols(4 * NSMALL)

    def p3_vreg(x, car2):
        bits = lax.bitcast_convert_type(x, jnp.int32)
        hi = bits >> 8
        addr = lanebase + (bits & (NSMALL - 1))
        plsc.addupdate_scatter(hist, [addr], ones, mask=hi == pre25)
        plsc.addupdate_scatter(hist, [addr + NSMALL], ones, mask=hi == pre50)
        plsc.addupdate_scatter(hist, [addr + 2 * NSMALL], ones,
                               mask=hi == pre75)
        return car2

    with scope("p3"):
        stream_pass(p3_vreg, 0, unroll=8)
    with scope("m3"):
        lane_reduce(3 * NSMALL)
    publish(3 * NSMALL)
    plsc.subcore_barrier()
    fetch(3 * NSMALL)
    ((b3_25, _), (b3_50, _), (b3_75, _)) = find_multi(
        NSMALL, (0, NSMALL, 2 * NSMALL), (r3_25, r3_50, r3_75))

    v25 = lax.bitcast_convert_type(
        jnp.broadcast_to((b25 << 19) | (b2_25 << 8) | b3_25, (16,)),
        jnp.float32)
    v50 = lax.bitcast_convert_type(
        jnp.broadcast_to((b50 << 19) | (b2_50 << 8) | b3_50, (16,)),
        jnp.float32)
    v75 = lax.bitcast_convert_type(
        jnp.broadcast_to((b75 << 19) | (b2_75 << 8) | b3_75, (16,)),
        jnp.float32)

    outv = jnp.where(lanes == 0, mn, jnp.zeros((16,), jnp.float32))
    outv = jnp.where(lanes == 1, v25, outv)
    outv = jnp.where(lanes == 2, v50, outv)
    outv = jnp.where(lanes == 3, v75, outv)
    outv = jnp.where(lanes == 4, mx, outv)
    vtmp[...] = outv

    @pl.when(half == 0)
    def _():
        pltpu.sync_copy(vtmp, out_hbm.at[img])


def _select_call(luma_p2, luma_t2):
    sel = functools.partial(
        pl.kernel,
        out_type=jax.ShapeDtypeStruct((16, 16), jnp.float32),
        mesh=plsc.VectorSubcoreMesh(core_axis_name="c", subcore_axis_name="s"),
        compiler_params=pltpu.CompilerParams(needs_layout_passes=False),
        scratch_types=[
            pltpu.VMEM((CHUNK,), jnp.float32),
            pltpu.VMEM((CHUNK,), jnp.float32),
            pltpu.VMEM((HISTW,), jnp.int32),
            pltpu.VMEM((STRIDE,), jnp.int32),
            pltpu.VMEM((16,), jnp.float32),
            pltpu.VMEM_SHARED((16, STRIDE), jnp.int32),
            pltpu.SemaphoreType.DMA,
            pltpu.SemaphoreType.DMA,
        ],
    )(_sc_select_body)
    return sel(luma_p2, luma_t2)


def _combine_body(sp_ref, st_ref, stats_ref, out_ref):
    inv_n = 1.0 / float(3 * 512 * 512)
    exposure = jnp.mean(jnp.abs(sp_ref[...] * inv_n - st_ref[...] * inv_n))
    st = stats_ref[...]
    d = jnp.abs(st[0:8, :] - st[8:16, :])
    lanemask = lax.broadcasted_iota(jnp.int32, (8, 16), 1) < 5
    hist = jnp.sum(jnp.where(lanemask, d, 0.0)) / 40.0
    out_ref[...] = jnp.full((1, 1), exposure + 0.5 * hist, jnp.float32)


def kernel(pred, target):
    luma_p, luma_t, sums_p, sums_t = pl.pallas_call(
        _luma_body,
        grid=(8,),
        in_specs=[
            pl.BlockSpec((1, 3, 512, 512), lambda i: (i, 0, 0, 0)),
            pl.BlockSpec((1, 3, 512, 512), lambda i: (i, 0, 0, 0)),
        ],
        out_specs=[
            pl.BlockSpec((1, 512, 512), lambda i: (i, 0, 0)),
            pl.BlockSpec((1, 512, 512), lambda i: (i, 0, 0)),
            pl.BlockSpec((1, 1, 128), lambda i: (i, 0, 0)),
            pl.BlockSpec((1, 1, 128), lambda i: (i, 0, 0)),
        ],
        out_shape=[
            jax.ShapeDtypeStruct((8, 512, 512), jnp.float32),
            jax.ShapeDtypeStruct((8, 512, 512), jnp.float32),
            jax.ShapeDtypeStruct((8, 1, 128), jnp.float32),
            jax.ShapeDtypeStruct((8, 1, 128), jnp.float32),
        ],
    )(pred, target)

    stats = _select_call(jnp.reshape(luma_p, (8, NELEM)),
                         jnp.reshape(luma_t, (8, NELEM)))

    out = pl.pallas_call(
        _combine_body,
        out_shape=jax.ShapeDtypeStruct((1, 1), jnp.float32),
    )(sums_p, sums_t, stats)
    return jnp.reshape(out, ())


# speedup vs baseline: 1.3314x; 1.1062x over previous
import functools

import jax
import jax.numpy as jnp
from jax import lax
from jax.experimental import pallas as pl
from jax.experimental.pallas import tpu as pltpu
from jax.experimental.pallas import tpu_sc as plsc

LANES = 16
NBIG = 2048
NSMALL = 256
STRIDE = 3 * NBIG
HISTW = LANES * STRIDE
NELEM = 512 * 512
HALFN = NELEM // 2
CHUNK = 8192
NCHUNK = HALFN // CHUNK
R25 = NELEM // 4 + 1
R50 = NELEM // 2 + 1
R75 = (3 * NELEM) // 4 + 1
UNROLL = 16


def _luma_body(p_ref, t_ref, lp_ref, lt_ref, sp_ref, st_ref):
    p0 = p_ref[0, 0]
    p1 = p_ref[0, 1]
    p2 = p_ref[0, 2]
    lp_ref[0] = 0.299 * p0 + 0.587 * p1 + 0.114 * p2
    sp = jnp.sum(p0) + jnp.sum(p1) + jnp.sum(p2)
    sp_ref[0, 0, :] = jnp.full((128,), sp, jnp.float32)
    t0 = t_ref[0, 0]
    t1 = t_ref[0, 1]
    t2 = t_ref[0, 2]
    lt_ref[0] = 0.299 * t0 + 0.587 * t1 + 0.114 * t2
    st = jnp.sum(t0) + jnp.sum(t1) + jnp.sum(t2)
    st_ref[0, 0, :] = jnp.full((128,), st, jnp.float32)


def _sc_select_body(lp_hbm, lt_hbm, out_hbm, buf0, buf1, hist, myflat,
                    vtmp, sh_hist, sem0, sem1):
    c = lax.axis_index("c")
    s = lax.axis_index("s")
    img = c * 8 + (s >> 1)
    half = s & 1
    part = s ^ 1
    lanes = lax.iota(jnp.int32, 16)
    ones = jnp.ones((16,), jnp.int32)
    zeros16 = jnp.zeros((16,), jnp.int32)
    lanebase = lanes * STRIDE
    base = half * HALFN

    def start_dma(ci, buf, sem):
        off = base + ci * CHUNK

        @pl.when(img < 8)
        def _():
            pltpu.async_copy(lp_hbm.at[img, pl.ds(off, CHUNK)], buf, sem)

        @pl.when(img >= 8)
        def _():
            pltpu.async_copy(lt_hbm.at[img - 8, pl.ds(off, CHUNK)], buf, sem)

    def wait_dma(buf, sem):
        pltpu.make_async_copy(lp_hbm.at[0, pl.ds(0, CHUNK)], buf, sem).wait()

    def stream_pass(per_vreg, car, unroll=UNROLL):
        car = jax.tree.map(jnp.asarray, car)
        def chunk_process(buf, c):
            def body(i, c2):
                x = buf[pl.ds(i, 16)]
                return per_vreg(x, c2)
            return plsc.parallel_loop(0, CHUNK, 16, unroll=unroll,
                                      carry=c)(body)

        def pair_body(i, c):
            ci0 = 2 * i
            wait_dma(buf0, sem0)
            start_dma(ci0 + 1, buf1, sem1)
            c = chunk_process(buf0, c)
            wait_dma(buf1, sem1)

            @pl.when(ci0 + 2 < NCHUNK)
            def _():
                start_dma(ci0 + 2, buf0, sem0)

            c = chunk_process(buf1, c)
            return c

        start_dma(0, buf0, sem0)
        return lax.fori_loop(0, NCHUNK // 2, pair_body, car)

    def zero_hist():
        def body(i):
            hist[pl.ds(i, 16)] = zeros16
        plsc.parallel_loop(0, HISTW, 16, unroll=8)(body)

    def zero_cols(ncols_pow2):
        sh = ncols_pow2.bit_length() - 1
        msk = ncols_pow2 - 1

        def body(i):
            hist[pl.ds(((i >> sh) * STRIDE) + (i & msk), 16)] = zeros16
        plsc.parallel_loop(0, LANES * ncols_pow2, 16, unroll=8)(body)

    def lane_reduce(ncols):
        def body(cv):
            acc = zeros16
            for l in range(LANES):
                acc = acc + hist[pl.ds(l * STRIDE + cv, 16)]
            myflat[pl.ds(cv, 16)] = acc
        plsc.parallel_loop(0, ncols, 16, unroll=2)(body)

    def publish(ncols):
        pltpu.sync_copy(myflat.at[pl.ds(0, ncols)],
                        sh_hist.at[s, pl.ds(0, ncols)])

    def fetch(ncols):
        pltpu.sync_copy(sh_hist.at[part, pl.ds(0, ncols)],
                        hist.at[pl.ds(0, ncols)])

    def find_shared(nbins, segoff, r1, r2, r3):
        def body(i, car):
            cum, st = car
            o = segoff + i * 16
            v = myflat[pl.ds(o, 16)] + hist[pl.ds(o, 16)]
            pc = plsc.cumsum(v) + cum
            new = []
            for (b, cb), r in zip(st, (r1, r2, r3)):
                lt = pc < r
                b = b + jnp.sum(jnp.where(lt, ones, zeros16))
                cb = jnp.maximum(cb, jnp.max(jnp.where(lt, pc, zeros16)))
                new.append((b, cb))
            return jnp.max(pc), tuple(new)
        z = jnp.int32(0)
        _, st = lax.fori_loop(0, nbins // 16, body,
                              (z, ((z, z), (z, z), (z, z))))
        return st

    def find_multi(nbins, segoffs, ranks, merged=True):
        def body(i, car):
            new = []
            for (cum, b, cb), o0, r in zip(car, segoffs, ranks):
                o = o0 + i * 16
                v = myflat[pl.ds(o, 16)]
                if merged:
                    v = v + hist[pl.ds(o, 16)]
                pc = plsc.cumsum(v) + cum
                lt = pc < r
                b = b + jnp.sum(jnp.where(lt, ones, zeros16))
                cb = jnp.maximum(cb, jnp.max(jnp.where(lt, pc, zeros16)))
                new.append((jnp.max(pc), b, cb))
            return tuple(new)
        z = jnp.int32(0)
        st = lax.fori_loop(0, nbins // 16, body, ((z,) * 3,) * 3)
        return tuple((b, cb) for _, b, cb in st)

    scope = jax.named_scope
    with scope("z1"):
        zero_cols(NBIG)

    def p1_vreg(x, car2):
        vmin, vmax = car2
        bits = lax.bitcast_convert_type(x, jnp.int32)
        plsc.addupdate_scatter(hist, [lanebase + (bits >> 19)], ones)
        return jnp.minimum(vmin, x), jnp.maximum(vmax, x)

    vmin0 = jnp.full((16,), jnp.inf, jnp.float32)
    vmax0 = jnp.full((16,), -jnp.inf, jnp.float32)
    with scope("p1"):
        vmin, vmax = stream_pass(p1_vreg, (vmin0, vmax0))
    mn = jnp.min(vmin)
    mx = jnp.max(vmax)

    with scope("m1"):
        lane_reduce(NBIG)
    mmv = jnp.where(lanes == 1, mx, mn)
    myflat[pl.ds(NBIG, 16)] = lax.bitcast_convert_type(mmv, jnp.int32)
    publish(NBIG + 128)
    plsc.subcore_barrier()
    fetch(NBIG + 128)
    pmm = lax.bitcast_convert_type(hist[pl.ds(NBIG, 16)], jnp.float32)
    mn = jnp.minimum(mn, jnp.min(pmm))
    mx = jnp.maximum(mx, jnp.max(pmm))
    plsc.subcore_barrier()

    with scope("f1"):
        ((b25, c25), (b50, c50), (b75, c75)) = find_shared(
            NBIG, 0, jnp.int32(R25), jnp.int32(R50), jnp.int32(R75))
    r2_25 = jnp.int32(R25) - c25
    r2_50 = jnp.int32(R50) - c50
    r2_75 = jnp.int32(R75) - c75

    CAP = 128
    sent = jnp.full((16,), 0x7F800000, jnp.int32)

    def sentinel_fill(seg):
        def body(i):
            myflat[pl.ds(seg + i, 16)] = sent
        plsc.parallel_loop(0, LANES * CAP, 16, unroll=8)(body)

    with scope("sf"):
        sentinel_fill(0)
    lanecap = lanes * CAP

    def collect_vreg(x, cnt):
        bits = lax.bitcast_convert_type(x, jnp.int32)
        top = bits >> 19
        m = (top == b25) | (top == b50) | (top == b75)
        plsc.store_scatter(myflat, [lanecap + cnt], bits, mask=m)
        return cnt + jnp.where(m, ones, zeros16)

    with scope("cp"):
        stream_pass(collect_vreg, zeros16, unroll=8)
    with scope("cm"):
        publish(LANES * CAP)
        plsc.subcore_barrier()
        pltpu.sync_copy(sh_hist.at[part, pl.ds(0, LANES * CAP)],
                        myflat.at[pl.ds(LANES * CAP, LANES * CAP)])

    with scope("z2"):
        zero_hist()

    def l2_scan(i):
        v = myflat[pl.ds(i, 16)]
        top = v >> 19
        addr = lanebase + ((v >> 8) & (NBIG - 1))
        plsc.addupdate_scatter(hist, [addr], ones, mask=top == b25)
        plsc.addupdate_scatter(hist, [addr + NBIG], ones, mask=top == b50)
        plsc.addupdate_scatter(hist, [addr + 2 * NBIG], ones, mask=top == b75)

    with scope("p2"):
        plsc.parallel_loop(0, 2 * LANES * CAP, 16, unroll=8)(l2_scan)
    with scope("m2"):
        lane_reduce(3 * NBIG)
    ((b2_25, c2_25), (b2_50, c2_50), (b2_75, c2_75)) = find_multi(
        NBIG, (0, NBIG, 2 * NBIG), (r2_25, r2_50, r2_75), merged=False)
    r3_25 = r2_25 - c2_25
    r3_50 = r2_50 - c2_50
    r3_75 = r2_75 - c2_75
    pre25 = (b25 << 11) | b2_25
    pre50 = (b50 << 11) | b2_50
    pre75 = (b75 << 11) | b2_75

    with scope("rf"):
        pltpu.sync_copy(sh_hist.at[s, pl.ds(0, LANES * CAP)],
                        myflat.at[pl.ds(0, LANES * CAP)])
        pltpu.sync_copy(sh_hist.at[part, pl.ds(0, LANES * CAP)],
                        myflat.at[pl.ds(LANES * CAP, LANES * CAP)])
    with scope("z3"):
        zero_cols(4 * NSMALL)

    def l3_scan(i):
        v = myflat[pl.ds(i, 16)]
        hi = v >> 8
        addr = lanebase + (v & (NSMALL - 1))
        plsc.addupdate_scatter(hist, [addr], ones, mask=hi == pre25)
        plsc.addupdate_scatter(hist, [addr + NSMALL], ones, mask=hi == pre50)
        plsc.addupdate_scatter(hist, [addr + 2 * NSMALL], ones,
                               mask=hi == pre75)

    with scope("p3"):
        plsc.parallel_loop(0, 2 * LANES * CAP, 16, unroll=8)(l3_scan)
    with scope("m3"):
        lane_reduce(3 * NSMALL)
    ((b3_25, _), (b3_50, _), (b3_75, _)) = find_multi(
        NSMALL, (0, NSMALL, 2 * NSMALL), (r3_25, r3_50, r3_75), merged=False)

    v25 = lax.bitcast_convert_type(
        jnp.broadcast_to((b25 << 19) | (b2_25 << 8) | b3_25, (16,)),
        jnp.float32)
    v50 = lax.bitcast_convert_type(
        jnp.broadcast_to((b50 << 19) | (b2_50 << 8) | b3_50, (16,)),
        jnp.float32)
    v75 = lax.bitcast_convert_type(
        jnp.broadcast_to((b75 << 19) | (b2_75 << 8) | b3_75, (16,)),
        jnp.float32)

    outv = jnp.where(lanes == 0, mn, jnp.zeros((16,), jnp.float32))
    outv = jnp.where(lanes == 1, v25, outv)
    outv = jnp.where(lanes == 2, v50, outv)
    outv = jnp.where(lanes == 3, v75, outv)
    outv = jnp.where(lanes == 4, mx, outv)
    vtmp[...] = outv

    @pl.when(half == 0)
    def _():
        pltpu.sync_copy(vtmp, out_hbm.at[img])


def _select_call(luma_p2, luma_t2):
    sel = functools.partial(
        pl.kernel,
        out_type=jax.ShapeDtypeStruct((16, 16), jnp.float32),
        mesh=plsc.VectorSubcoreMesh(core_axis_name="c", subcore_axis_name="s"),
        compiler_params=pltpu.CompilerParams(needs_layout_passes=False),
        scratch_types=[
            pltpu.VMEM((CHUNK,), jnp.float32),
            pltpu.VMEM((CHUNK,), jnp.float32),
            pltpu.VMEM((HISTW,), jnp.int32),
            pltpu.VMEM((STRIDE,), jnp.int32),
            pltpu.VMEM((16,), jnp.float32),
            pltpu.VMEM_SHARED((16, STRIDE), jnp.int32),
            pltpu.SemaphoreType.DMA,
            pltpu.SemaphoreType.DMA,
        ],
    )(_sc_select_body)
    return sel(luma_p2, luma_t2)


def _combine_body(sp_ref, st_ref, stats_ref, out_ref):
    inv_n = 1.0 / float(3 * 512 * 512)
    exposure = jnp.mean(jnp.abs(sp_ref[...] * inv_n - st_ref[...] * inv_n))
    st = stats_ref[...]
    d = jnp.abs(st[0:8, :] - st[8:16, :])
    lanemask = lax.broadcasted_iota(jnp.int32, (8, 16), 1) < 5
    hist = jnp.sum(jnp.where(lanemask, d, 0.0)) / 40.0
    out_ref[...] = jnp.full((1, 1), exposure + 0.5 * hist, jnp.float32)


def kernel(pred, target):
    luma_p, luma_t, sums_p, sums_t = pl.pallas_call(
        _luma_body,
        grid=(8,),
        in_specs=[
            pl.BlockSpec((1, 3, 512, 512), lambda i: (i, 0, 0, 0)),
            pl.BlockSpec((1, 3, 512, 512), lambda i: (i, 0, 0, 0)),
        ],
        out_specs=[
            pl.BlockSpec((1, 512, 512), lambda i: (i, 0, 0)),
            pl.BlockSpec((1, 512, 512), lambda i: (i, 0, 0)),
            pl.BlockSpec((1, 1, 128), lambda i: (i, 0, 0)),
            pl.BlockSpec((1, 1, 128), lambda i: (i, 0, 0)),
        ],
        out_shape=[
            jax.ShapeDtypeStruct((8, 512, 512), jnp.float32),
            jax.ShapeDtypeStruct((8, 512, 512), jnp.float32),
            jax.ShapeDtypeStruct((8, 1, 128), jnp.float32),
            jax.ShapeDtypeStruct((8, 1, 128), jnp.float32),
        ],
    )(pred, target)

    stats = _select_call(jnp.reshape(luma_p, (8, NELEM)),
                         jnp.reshape(luma_t, (8, NELEM)))

    out = pl.pallas_call(
        _combine_body,
        out_shape=jax.ShapeDtypeStruct((1, 1), jnp.float32),
    )(sums_p, sums_t, stats)
    return jnp.reshape(out, ())
